# baseline (speedup 1.0000x reference)
import functools

import jax
import jax.numpy as jnp
from jax import lax
from jax.experimental import pallas as pl
from jax.experimental.pallas import tpu as pltpu
from jax.experimental.pallas import tpu_sc as plsc

_N = 10000
_E = 320000
_D = 128
_EPS = 1e-7
_MIN = 1e-15
_MAX = 1e6
_SQ50 = 7.0710678118654755

_NB = 1000
_EB = 16000
_CH = 32
_EPW = _E // 32


def _fsqrt(x):
    i = lax.bitcast_convert_type(x, jnp.int32)
    y = lax.bitcast_convert_type(jnp.int32(0x5F3759DF) - (i >> 1), jnp.float32)
    for _ in range(3):
        y = y * (1.5 - 0.5 * x * y * y)
    return x * y


def _flog(z):
    i = lax.bitcast_convert_type(z, jnp.int32)
    e = (i >> 23) - 127
    m = lax.bitcast_convert_type((i & 0x007FFFFF) | 0x3F800000, jnp.float32)
    big = m > 1.4142135
    m = jnp.where(big, 0.5 * m, m)
    ef = (e + big.astype(jnp.int32)).astype(jnp.float32)
    s = (m - 1.0) / (m + 1.0)
    s2 = s * s
    p = 2.0 * s * (1.0 + s2 * (1.0 / 3.0 + s2 * (0.2 + s2 * (1.0 / 7.0 + s2 * (1.0 / 9.0)))))
    return ef * 0.6931471805599453 + p


def _node_body(h_ref, g_ref, b_ref, w_ref, linb_ref, w1r_ref, w1c_ref, b1_ref,
               trow_ref, tcol_ref, hp_ref):
    X = h_ref[...]
    iot = lax.broadcasted_iota(jnp.int32, X.shape, 1)
    sp = iot > 0

    def spat(v):
        return jnp.where(sp, v, 0.0)

    def arcosh(x):
        return jnp.log(x + jnp.sqrt(jnp.clip(x * x - 1.0, 1e-15, None)))

    def logmap0(x):
        y = spat(x)
        yn = jnp.maximum(jnp.sqrt(jnp.sum(y * y, 1, keepdims=True)), _MIN)
        th = jnp.maximum(x[:, 0:1], 1.0 + _EPS)
        return arcosh(th) * y / yn

    def sinh(t):
        return 0.5 * (jnp.exp(t) - jnp.exp(-t))

    def cosh(t):
        return 0.5 * (jnp.exp(t) + jnp.exp(-t))

    def expmap0p(u):
        y = spat(u)
        yn = jnp.maximum(jnp.sqrt(jnp.sum(y * y, 1, keepdims=True)), _MIN)
        xr = sinh(yn) * y / yn
        x0 = jnp.sqrt(jnp.clip(1.0 + jnp.sum(xr * xr, 1, keepdims=True), _EPS, None))
        return jnp.where(iot == 0, x0, xr)

    def dotT(a, m_ref):
        return lax.dot_general(a, m_ref[...], (((1,), (1,)), ((), ())),
                               preferred_element_type=jnp.float32)

    ht = logmap0(X)
    mu = jnp.sum(ht, 1, keepdims=True) / (_D - 1)
    xc = ht - mu
    var = jnp.sum(spat(xc * xc), 1, keepdims=True) / (_D - 1)
    ln = xc / jnp.sqrt(var + 1e-5) * g_ref[...] + b_ref[...]
    h1 = expmap0p(spat(ln))
    u = logmap0(h1)
    h2 = expmap0p(dotT(u, w_ref))
    hb = expmap0p(spat(linb_ref[...]))
    ub = logmap0(hb)
    x0 = h2[:, 0:1]
    y = spat(h2)
    yn = jnp.maximum(jnp.sqrt(jnp.sum(y * y, 1, keepdims=True)), _MIN)
    yu = y / yn
    vv = jnp.where(iot == 0, -yn, (1.0 - x0) * yu)
    alpha = jnp.sum(yu * ub, 1, keepdims=True)
    w = ub - alpha * vv
    ux = jnp.sum(y * spat(w), 1, keepdims=True)
    w0 = ux / jnp.maximum(x0, _EPS)
    v = jnp.where(iot == 0, w0, w)
    mdv = jnp.sum(v * v, 1, keepdims=True) - 2.0 * w0 * w0
    normu = jnp.minimum(jnp.sqrt(jnp.clip(mdv, _EPS, None)), _MAX)
    th2 = jnp.maximum(normu, _MIN)
    rsp = cosh(th2) * y + sinh(th2) * spat(v) / th2
    x0p = jnp.sqrt(jnp.clip(1.0 + jnp.sum(rsp * rsp, 1, keepdims=True), _EPS, None))
    hp = jnp.where(iot == 0, x0p, rsp)
    xt = logmap0(hp)
    ar = dotT(xt, w1r_ref) + b1_ref[...]
    ac = dotT(xt, w1c_ref)
    trow_ref[...] = ar.astype(jnp.bfloat16)
    tcol_ref[...] = ac.astype(jnp.bfloat16)
    hp_ref[...] = hp


def _stage1(h, g128, b128, lin_W, linb, w1r, w1c, b1):
    grid = _N // _NB
    full = lambda shape: pl.BlockSpec(shape, lambda i: (0, 0))
    return pl.pallas_call(
        _node_body,
        grid=(grid,),
        in_specs=[
            pl.BlockSpec((_NB, _D), lambda i: (i, 0)),
            full((1, _D)), full((1, _D)), full((_D, _D)), full((1, _D)),
            full((_D, _D)), full((_D, _D)), full((1, _D)),
        ],
        out_specs=[
            pl.BlockSpec((_NB, _D), lambda i: (i, 0)),
            pl.BlockSpec((_NB, _D), lambda i: (i, 0)),
            pl.BlockSpec((_NB, _D), lambda i: (i, 0)),
        ],
        out_shape=[
            jax.ShapeDtypeStruct((_N, _D), jnp.bfloat16),
            jax.ShapeDtypeStruct((_N, _D), jnp.bfloat16),
            jax.ShapeDtypeStruct((_N, _D), jnp.float32),
        ],
    )(h, g128, b128, lin_W, linb, w1r, w1c, b1)


def _ee_body(eat_ref, w1e_ref, ee_ref):
    eat = eat_ref[...]
    w = w1e_ref[...]
    ee_ref[...] = lax.dot_general(eat, w, (((0,), (0,)), ((), ())),
                                  preferred_element_type=jnp.float32
                                  ).astype(jnp.bfloat16)


def _stage_ee(edge_attrT, w1eT):
    grid = _E // _EB
    return pl.pallas_call(
        _ee_body,
        grid=(grid,),
        in_specs=[
            pl.BlockSpec((4, _EB), lambda i: (0, i)),
            pl.BlockSpec((4, _D), lambda i: (0, 0)),
        ],
        out_specs=pl.BlockSpec((_EB, _D), lambda i: (i, 0)),
        out_shape=jax.ShapeDtypeStruct((_E, _D), jnp.bfloat16),
    )(edge_attrT, w1eT)


def _edge_sc(row, col, htab, arow, acol, ee, w2, b2x, zeros):
    mesh = plsc.VectorSubcoreMesh(core_axis_name="c", subcore_axis_name="s")
    rows_per_tile = (_N // 16) // 8 * 8
    tail0 = 16 * rows_per_tile
    tailn = _N - tail0

    @functools.partial(
        pl.kernel,
        out_type=jax.ShapeDtypeStruct((2, _N, _D), jnp.float32),
        mesh=mesh,
        scratch_types=[
            pltpu.VMEM((2, _CH), jnp.int32),
            pltpu.VMEM((2, _CH), jnp.int32),
            pltpu.VMEM((2, _CH, _D), jnp.float32),
            pltpu.VMEM((2, _CH, _D), jnp.float32),
            pltpu.VMEM((2, _CH, _D), jnp.bfloat16),
            pltpu.VMEM((2, _CH, _D), jnp.bfloat16),
            pltpu.VMEM((2, _CH, _D), jnp.bfloat16),
            pltpu.VMEM((_CH, _D), jnp.float32),
            pltpu.VMEM((_D,), jnp.bfloat16),
            pltpu.VMEM((16,), jnp.float32),
            pltpu.VMEM_SHARED((_N, _D), jnp.float32),
            pltpu.SemaphoreType.DMA,
            pltpu.SemaphoreType.DMA,
        ],
        compiler_params=pltpu.CompilerParams(use_tc_tiling_on_sc=False,
                                             needs_layout_passes=False),
    )
    def k(row_hbm, col_hbm, h_hbm, ar_hbm, ac_hbm, ee_hbm, w2_hbm, b2_hbm,
          z_hbm, out_hbm, rowv, colv, xbuf, ybuf, arbuf, acbuf, eebuf,
          contrib, w2v, b2v, aggsh, sem0, sem1):
        c = lax.axis_index("c")
        s = lax.axis_index("s")
        widx = c * 16 + s
        r0 = s * rows_per_tile
        pltpu.sync_copy(z_hbm.at[pl.ds(r0, rows_per_tile)],
                        aggsh.at[pl.ds(r0, rows_per_tile)])

        if tailn:
            @pl.when(s == 0)
            def _():
                pltpu.sync_copy(z_hbm.at[pl.ds(tail0, tailn)],
                                aggsh.at[pl.ds(tail0, tailn)])

        pltpu.sync_copy(w2_hbm, w2v)
        pltpu.sync_copy(b2_hbm, b2v)
        plsc.subcore_barrier()
        iota = lax.iota(jnp.int32, 16)
        lane0 = (iota == 0).astype(jnp.float32)
        b2vec = b2v[...]
        w2reg = [plsc.unpack(w2v[pl.ds(32 * k, 32)],
                             format=plsc.PackFormat.INTERLEAVED)
                 for k in range(_D // 32)]
        K8 = _D // 16

        def bc(sc):
            return jnp.full((16,), sc, jnp.float32)

        def rsum(v):
            return jnp.sum(v, axis=0)

        sems = (sem0, sem1)
        nfull = _EPW // _CH

        def base_of(ci):
            return widx * (nfull * _CH) + ci * _CH

        def issue(base, p):
            pltpu.sync_copy(row_hbm.at[pl.ds(base, _CH)], rowv.at[p])
            pltpu.sync_copy(col_hbm.at[pl.ds(base, _CH)], colv.at[p])
            pltpu.async_copy(h_hbm.at[rowv.at[p]], xbuf.at[p], sems[p])
            pltpu.async_copy(h_hbm.at[colv.at[p]], ybuf.at[p], sems[p])
            pltpu.async_copy(ar_hbm.at[rowv.at[p]], arbuf.at[p], sems[p])
            pltpu.async_copy(ac_hbm.at[colv.at[p]], acbuf.at[p], sems[p])
            pltpu.async_copy(ee_hbm.at[pl.ds(base, _CH)], eebuf.at[p], sems[p])

        def drain(base, p):
            pltpu.make_async_copy(h_hbm.at[rowv.at[p]], xbuf.at[p], sems[p]).wait()
            pltpu.make_async_copy(h_hbm.at[colv.at[p]], ybuf.at[p], sems[p]).wait()
            pltpu.make_async_copy(ar_hbm.at[rowv.at[p]], arbuf.at[p], sems[p]).wait()
            pltpu.make_async_copy(ac_hbm.at[colv.at[p]], acbuf.at[p], sems[p]).wait()
            pltpu.make_async_copy(ee_hbm.at[pl.ds(base, _CH)], eebuf.at[p], sems[p]).wait()

        def compute(p):
            @functools.partial(plsc.parallel_loop, 0, _CH, unroll=4)
            def _(e):
                xs = [xbuf[p, e, pl.ds(16 * k, 16)] for k in range(K8)]
                ys = [ybuf[p, e, pl.ds(16 * k, 16)] for k in range(K8)]
                dv = xs[0] * ys[0]
                for k in range(1, K8):
                    dv = dv + xs[k] * ys[k]
                s1 = bc(rsum(dv))
                x0 = bc(rsum(xs[0] * lane0))
                y0 = bc(rsum(ys[0] * lane0))
                pv = None
                for k in range(_D // 32):
                    pre = arbuf[p, e, pl.ds(32 * k, 32)] \
                        + acbuf[p, e, pl.ds(32 * k, 32)] \
                        + eebuf[p, e, pl.ds(32 * k, 32)]
                    pa, pb = plsc.unpack(pre, format=plsc.PackFormat.INTERLEAVED)
                    for ph, wh in ((pa, w2reg[k][0]), (pb, w2reg[k][1])):
                        sa = ph / (1.0 + jnp.exp(-ph))
                        pv = sa * wh if pv is None else pv + sa * wh
                a2 = bc(rsum(pv))
                md = s1 - 2.0 * x0 * y0
                xy = jnp.minimum(md + 1.0, -_EPS) - 1.0
                th = jnp.maximum(-md, 1.0 + _EPS)
                dist = jnp.minimum(_flog(th + _fsqrt((th - 1.0) * (th + 1.0))), _SQ50)
                us = [ys[k] + xy * xs[k] for k in range(K8)]
                u0 = y0 + xy * x0
                qv = us[0] * us[0]
                tv = xs[0] * us[0]
                for k in range(1, K8):
                    qv = qv + us[k] * us[k]
                    tv = tv + xs[k] * us[k]
                s2 = bc(rsum(qv))
                s3 = bc(rsum(tv))
                normu = _fsqrt(jnp.maximum(s2 - 2.0 * u0 * u0, _EPS))
                att = 1.0 / (1.0 + jnp.exp(-(a2 + b2vec)))
                coef = att * dist / normu
                o0 = coef * (s3 - x0 * u0) / x0
                first = coef * us[0]
                contrib[e, pl.ds(0, 16)] = jnp.where(iota == 0, o0, first)
                for k in range(1, K8):
                    contrib[e, pl.ds(16 * k, 16)] = coef * us[k]

            pltpu.sync_copy(contrib, aggsh.at[rowv.at[p]], add=True)

        main_span = 32 * nfull * _CH

        issue(base_of(0), 0)

        def outer(g, _):
            c0 = 2 * g
            issue(base_of(c0 + 1), 1)
            drain(base_of(c0), 0)
            compute(0)

            @pl.when(g < nfull // 2 - 1)
            def _():
                issue(base_of(c0 + 2), 0)

            drain(base_of(c0 + 1), 1)
            compute(1)
            return 0

        lax.fori_loop(0, nfull // 2, outer, 0)

        @pl.when(widx < (_E - main_span) // _CH)
        def _():
            base = main_span + widx * _CH
            issue(base, 0)
            drain(base, 0)
            compute(0)

        plsc.subcore_barrier()
        pltpu.sync_copy(aggsh.at[pl.ds(r0, rows_per_tile)],
                        out_hbm.at[c, pl.ds(r0, rows_per_tile)])

        if tailn:
            @pl.when(s == 0)
            def _():
                pltpu.sync_copy(aggsh.at[pl.ds(tail0, tailn)],
                                out_hbm.at[c, pl.ds(tail0, tailn)])

    return k(row, col, htab, arow, acol, ee, w2, b2x, zeros)


def _final_body(hp_ref, p_ref, out_ref):
    x = hp_ref[...]
    p = p_ref[...]
    agg = p[0] + p[1]
    iot = lax.broadcasted_iota(jnp.int32, x.shape, 1)
    sp = iot > 0

    def spat(v):
        return jnp.where(sp, v, 0.0)

    y = spat(x)
    a0 = agg[:, 0:1]
    mdv = jnp.sum(agg * agg, 1, keepdims=True) - 2.0 * a0 * a0
    normu = jnp.minimum(jnp.sqrt(jnp.clip(mdv, _EPS, None)), _MAX)
    th = jnp.maximum(normu, _MIN)
    rsp = (0.5 * (jnp.exp(th) + jnp.exp(-th))) * y \
        + (0.5 * (jnp.exp(th) - jnp.exp(-th))) * spat(agg) / th
    x0p = jnp.sqrt(jnp.clip(1.0 + jnp.sum(rsp * rsp, 1, keepdims=True), _EPS, None))
    yn = jnp.maximum(jnp.sqrt(jnp.sum(rsp * rsp, 1, keepdims=True)), _MIN)
    thh = jnp.maximum(x0p, 1.0 + _EPS)
    ach = jnp.log(thh + jnp.sqrt(jnp.clip(thh * thh - 1.0, 1e-15, None)))
    lt = ach * rsp / yn
    st = spat(lt / (1.0 + jnp.exp(-lt)))
    sn = jnp.maximum(jnp.sqrt(jnp.sum(st * st, 1, keepdims=True)), _MIN)
    xr = (0.5 * (jnp.exp(sn) - jnp.exp(-sn))) * st / sn
    ox0 = jnp.sqrt(jnp.clip(1.0 + jnp.sum(xr * xr, 1, keepdims=True), _EPS, None))
    out_ref[...] = jnp.where(iot == 0, ox0, xr)


def _stage3(hp, parts):
    grid = _N // _NB
    bs = pl.BlockSpec((_NB, _D), lambda i: (i, 0))
    return pl.pallas_call(
        _final_body,
        grid=(grid,),
        in_specs=[bs, pl.BlockSpec((2, _NB, _D), lambda i: (0, i, 0))],
        out_specs=bs,
        out_shape=jax.ShapeDtypeStruct((_N, _D), jnp.float32),
    )(hp, parts)


def kernel(h, edge_index, edge_attr, ln_g, ln_b, lin_W, lin_b,
           att_W1, att_b1, att_W2, att_b2):
    g128 = jnp.concatenate([jnp.ones((1,), jnp.float32), ln_g])[None, :]
    b128 = jnp.concatenate([jnp.zeros((1,), jnp.float32), ln_b])[None, :]
    linb = lin_b[None, :]
    w1r = att_W1[:, :_D]
    w1c = att_W1[:, _D:2 * _D]
    w1eT = att_W1[:, 2 * _D:].T
    b1 = att_b1[None, :]
    arow, acol, hp = _stage1(h, g128, b128, lin_W, linb, w1r, w1c, b1)
    ee = _stage_ee(edge_attr.T, w1eT)
    row = edge_index[0].astype(jnp.int32)
    col = edge_index[1].astype(jnp.int32)
    w2 = att_W2.reshape(_D).astype(jnp.bfloat16)
    b2x = jnp.ones((16,), jnp.float32) * att_b2
    zeros = jnp.zeros((_N, _D), jnp.float32)
    parts = _edge_sc(row, col, hp, arow, acol, ee, w2, b2x, zeros)
    return _stage3(hp, parts)


# speedup vs baseline: 1.2748x; 1.2748x over previous
import functools

import jax
import jax.numpy as jnp
from jax import lax
from jax.experimental import pallas as pl
from jax.experimental.pallas import tpu as pltpu
from jax.experimental.pallas import tpu_sc as plsc

_N = 10000
_E = 320000
_D = 128
_EPS = 1e-7
_MIN = 1e-15
_MAX = 1e6
_SQ50 = 7.0710678118654755

_NB = 1000
_EB = 16000
_CH = 32
_EPW = _E // 32


def _fsqrt(x):
    i = lax.bitcast_convert_type(x, jnp.int32)
    y = lax.bitcast_convert_type(jnp.int32(0x5F3759DF) - (i >> 1), jnp.float32)
    for _ in range(3):
        y = y * (1.5 - 0.5 * x * y * y)
    return x * y


def _flog(z):
    i = lax.bitcast_convert_type(z, jnp.int32)
    e = (i >> 23) - 127
    m = lax.bitcast_convert_type((i & 0x007FFFFF) | 0x3F800000, jnp.float32)
    big = m > 1.4142135
    m = jnp.where(big, 0.5 * m, m)
    ef = (e + big.astype(jnp.int32)).astype(jnp.float32)
    s = (m - 1.0) / (m + 1.0)
    s2 = s * s
    p = 2.0 * s * (1.0 + s2 * (1.0 / 3.0 + s2 * (0.2 + s2 * (1.0 / 7.0 + s2 * (1.0 / 9.0)))))
    return ef * 0.6931471805599453 + p


def _node_body(h_ref, g_ref, b_ref, w_ref, linb_ref, w1r_ref, w1c_ref, b1_ref,
               trow_ref, tcol_ref, hp_ref):
    X = h_ref[...]
    iot = lax.broadcasted_iota(jnp.int32, X.shape, 1)
    sp = iot > 0

    def spat(v):
        return jnp.where(sp, v, 0.0)

    def arcosh(x):
        return jnp.log(x + jnp.sqrt(jnp.clip(x * x - 1.0, 1e-15, None)))

    def logmap0(x):
        y = spat(x)
        yn = jnp.maximum(jnp.sqrt(jnp.sum(y * y, 1, keepdims=True)), _MIN)
        th = jnp.maximum(x[:, 0:1], 1.0 + _EPS)
        return arcosh(th) * y / yn

    def sinh(t):
        return 0.5 * (jnp.exp(t) - jnp.exp(-t))

    def cosh(t):
        return 0.5 * (jnp.exp(t) + jnp.exp(-t))

    def expmap0p(u):
        y = spat(u)
        yn = jnp.maximum(jnp.sqrt(jnp.sum(y * y, 1, keepdims=True)), _MIN)
        xr = sinh(yn) * y / yn
        x0 = jnp.sqrt(jnp.clip(1.0 + jnp.sum(xr * xr, 1, keepdims=True), _EPS, None))
        return jnp.where(iot == 0, x0, xr)

    def dotT(a, m_ref):
        return lax.dot_general(a, m_ref[...], (((1,), (1,)), ((), ())),
                               preferred_element_type=jnp.float32)

    ht = logmap0(X)
    mu = jnp.sum(ht, 1, keepdims=True) / (_D - 1)
    xc = ht - mu
    var = jnp.sum(spat(xc * xc), 1, keepdims=True) / (_D - 1)
    ln = xc / jnp.sqrt(var + 1e-5) * g_ref[...] + b_ref[...]
    h1 = expmap0p(spat(ln))
    u = logmap0(h1)
    h2 = expmap0p(dotT(u, w_ref))
    hb = expmap0p(spat(linb_ref[...]))
    ub = logmap0(hb)
    x0 = h2[:, 0:1]
    y = spat(h2)
    yn = jnp.maximum(jnp.sqrt(jnp.sum(y * y, 1, keepdims=True)), _MIN)
    yu = y / yn
    vv = jnp.where(iot == 0, -yn, (1.0 - x0) * yu)
    alpha = jnp.sum(yu * ub, 1, keepdims=True)
    w = ub - alpha * vv
    ux = jnp.sum(y * spat(w), 1, keepdims=True)
    w0 = ux / jnp.maximum(x0, _EPS)
    v = jnp.where(iot == 0, w0, w)
    mdv = jnp.sum(v * v, 1, keepdims=True) - 2.0 * w0 * w0
    normu = jnp.minimum(jnp.sqrt(jnp.clip(mdv, _EPS, None)), _MAX)
    th2 = jnp.maximum(normu, _MIN)
    rsp = cosh(th2) * y + sinh(th2) * spat(v) / th2
    x0p = jnp.sqrt(jnp.clip(1.0 + jnp.sum(rsp * rsp, 1, keepdims=True), _EPS, None))
    hp = jnp.where(iot == 0, x0p, rsp)
    xt = logmap0(hp)
    ar = dotT(xt, w1r_ref) + b1_ref[...]
    ac = dotT(xt, w1c_ref)
    trow_ref[...] = jnp.concatenate([hp, ar], axis=1)
    tcol_ref[...] = jnp.concatenate([hp, ac], axis=1)
    hp_ref[...] = hp


def _stage1(h, g128, b128, lin_W, linb, w1r, w1c, b1):
    grid = _N // _NB
    full = lambda shape: pl.BlockSpec(shape, lambda i: (0, 0))
    return pl.pallas_call(
        _node_body,
        grid=(grid,),
        in_specs=[
            pl.BlockSpec((_NB, _D), lambda i: (i, 0)),
            full((1, _D)), full((1, _D)), full((_D, _D)), full((1, _D)),
            full((_D, _D)), full((_D, _D)), full((1, _D)),
        ],
        out_specs=[
            pl.BlockSpec((_NB, 2 * _D), lambda i: (i, 0)),
            pl.BlockSpec((_NB, 2 * _D), lambda i: (i, 0)),
            pl.BlockSpec((_NB, _D), lambda i: (i, 0)),
        ],
        out_shape=[
            jax.ShapeDtypeStruct((_N, 2 * _D), jnp.float32),
            jax.ShapeDtypeStruct((_N, 2 * _D), jnp.float32),
            jax.ShapeDtypeStruct((_N, _D), jnp.float32),
        ],
    )(h, g128, b128, lin_W, linb, w1r, w1c, b1)


def _ee_body(eat_ref, w1e_ref, ee_ref):
    eat = eat_ref[...]
    w = w1e_ref[...]
    ee_ref[...] = lax.dot_general(eat, w, (((0,), (0,)), ((), ())),
                                  preferred_element_type=jnp.float32)


def _stage_ee(edge_attrT, w1eT):
    grid = _E // _EB
    return pl.pallas_call(
        _ee_body,
        grid=(grid,),
        in_specs=[
            pl.BlockSpec((4, _EB), lambda i: (0, i)),
            pl.BlockSpec((4, _D), lambda i: (0, 0)),
        ],
        out_specs=pl.BlockSpec((_EB, _D), lambda i: (i, 0)),
        out_shape=jax.ShapeDtypeStruct((_E, _D), jnp.float32),
    )(edge_attrT, w1eT)


def _edge_sc(row, col, trow, tcol, ee, w2, b2x, zeros):
    mesh = plsc.VectorSubcoreMesh(core_axis_name="c", subcore_axis_name="s")
    rows_per_tile = (_N // 16) // 8 * 8
    tail0 = 16 * rows_per_tile
    tailn = _N - tail0

    @functools.partial(
        pl.kernel,
        out_type=jax.ShapeDtypeStruct((2, _N, _D), jnp.float32),
        mesh=mesh,
        scratch_types=[
            pltpu.VMEM((2, _CH), jnp.int32),
            pltpu.VMEM((2, _CH), jnp.int32),
            pltpu.VMEM((2, _CH, 2 * _D), jnp.float32),
            pltpu.VMEM((2, _CH, 2 * _D), jnp.float32),
            pltpu.VMEM((2, _CH, _D), jnp.float32),
            pltpu.VMEM((_CH, _D), jnp.float32),
            pltpu.VMEM((_D,), jnp.float32),
            pltpu.VMEM((16,), jnp.float32),
            pltpu.VMEM_SHARED((_N, _D), jnp.float32),
            pltpu.SemaphoreType.DMA,
            pltpu.SemaphoreType.DMA,
        ],
        compiler_params=pltpu.CompilerParams(use_tc_tiling_on_sc=False,
                                             needs_layout_passes=False),
    )
    def k(row_hbm, col_hbm, trow_hbm, tcol_hbm, ee_hbm, w2_hbm, b2_hbm, z_hbm,
          out_hbm, rowv, colv, rbuf, cbuf, eebuf, contrib, w2v, b2v, aggsh,
          sem0, sem1):
        c = lax.axis_index("c")
        s = lax.axis_index("s")
        widx = c * 16 + s
        r0 = s * rows_per_tile
        pltpu.sync_copy(z_hbm.at[pl.ds(r0, rows_per_tile)],
                        aggsh.at[pl.ds(r0, rows_per_tile)])

        if tailn:
            @pl.when(s == 0)
            def _():
                pltpu.sync_copy(z_hbm.at[pl.ds(tail0, tailn)],
                                aggsh.at[pl.ds(tail0, tailn)])

        pltpu.sync_copy(w2_hbm, w2v)
        pltpu.sync_copy(b2_hbm, b2v)
        plsc.subcore_barrier()
        iota = lax.iota(jnp.int32, 16)
        lane0 = (iota == 0).astype(jnp.float32)
        b2vec = b2v[...]
        w2reg = [w2v[pl.ds(16 * k, 16)] for k in range(8)]
        K8 = _D // 16

        def bc(sc):
            return jnp.full((16,), sc, jnp.float32)

        def rsum(v):
            return jnp.sum(v, axis=0)

        sems = (sem0, sem1)
        nfull = _EPW // _CH

        def base_of(ci):
            return widx * (nfull * _CH) + ci * _CH

        def issue(base, p):
            pltpu.sync_copy(row_hbm.at[pl.ds(base, _CH)], rowv.at[p])
            pltpu.sync_copy(col_hbm.at[pl.ds(base, _CH)], colv.at[p])
            pltpu.async_copy(trow_hbm.at[rowv.at[p]], rbuf.at[p], sems[p])
            pltpu.async_copy(tcol_hbm.at[colv.at[p]], cbuf.at[p], sems[p])
            pltpu.async_copy(ee_hbm.at[pl.ds(base, _CH)], eebuf.at[p], sems[p])

        def drain(base, p):
            pltpu.make_async_copy(trow_hbm.at[rowv.at[p]], rbuf.at[p], sems[p]).wait()
            pltpu.make_async_copy(tcol_hbm.at[colv.at[p]], cbuf.at[p], sems[p]).wait()
            pltpu.make_async_copy(ee_hbm.at[pl.ds(base, _CH)], eebuf.at[p], sems[p]).wait()

        def compute(p):
            @functools.partial(plsc.parallel_loop, 0, _CH, unroll=4)
            def _(e):
                xs = [rbuf[p, e, pl.ds(16 * k, 16)] for k in range(K8)]
                ys = [cbuf[p, e, pl.ds(16 * k, 16)] for k in range(K8)]
                dv = xs[0] * ys[0]
                for k in range(1, K8):
                    dv = dv + xs[k] * ys[k]
                s1 = bc(rsum(dv))
                x0 = bc(rsum(xs[0] * lane0))
                y0 = bc(rsum(ys[0] * lane0))
                pv = None
                for k in range(K8):
                    pre = rbuf[p, e, pl.ds(_D + 16 * k, 16)] \
                        + cbuf[p, e, pl.ds(_D + 16 * k, 16)] \
                        + eebuf[p, e, pl.ds(16 * k, 16)]
                    sa = pre / (1.0 + jnp.exp(-pre))
                    pv = sa * w2reg[k] if pv is None else pv + sa * w2reg[k]
                a2 = bc(rsum(pv))
                md = s1 - 2.0 * x0 * y0
                xy = jnp.minimum(md + 1.0, -_EPS) - 1.0
                th = jnp.maximum(-md, 1.0 + _EPS)
                dist = jnp.minimum(_flog(th + _fsqrt((th - 1.0) * (th + 1.0))), _SQ50)
                us = [ys[k] + xy * xs[k] for k in range(K8)]
                u0 = y0 + xy * x0
                qv = us[0] * us[0]
                tv = xs[0] * us[0]
                for k in range(1, K8):
                    qv = qv + us[k] * us[k]
                    tv = tv + xs[k] * us[k]
                s2 = bc(rsum(qv))
                s3 = bc(rsum(tv))
                normu = _fsqrt(jnp.maximum(s2 - 2.0 * u0 * u0, _EPS))
                att = 1.0 / (1.0 + jnp.exp(-(a2 + b2vec)))
                coef = att * dist / normu
                o0 = coef * (s3 - x0 * u0) / x0
                first = coef * us[0]
                contrib[e, pl.ds(0, 16)] = jnp.where(iota == 0, o0, first)
                for k in range(1, K8):
                    contrib[e, pl.ds(16 * k, 16)] = coef * us[k]

            pltpu.sync_copy(contrib, aggsh.at[rowv.at[p]], add=True)

        main_span = 32 * nfull * _CH

        issue(base_of(0), 0)

        def outer(g, _):
            c0 = 2 * g
            issue(base_of(c0 + 1), 1)
            drain(base_of(c0), 0)
            compute(0)

            @pl.when(g < nfull // 2 - 1)
            def _():
                issue(base_of(c0 + 2), 0)

            drain(base_of(c0 + 1), 1)
            compute(1)
            return 0

        lax.fori_loop(0, nfull // 2, outer, 0)

        @pl.when(widx < (_E - main_span) // _CH)
        def _():
            base = main_span + widx * _CH
            issue(base, 0)
            drain(base, 0)
            compute(0)

        plsc.subcore_barrier()
        pltpu.sync_copy(aggsh.at[pl.ds(r0, rows_per_tile)],
                        out_hbm.at[c, pl.ds(r0, rows_per_tile)])

        if tailn:
            @pl.when(s == 0)
            def _():
                pltpu.sync_copy(aggsh.at[pl.ds(tail0, tailn)],
                                out_hbm.at[c, pl.ds(tail0, tailn)])

    return k(row, col, trow, tcol, ee, w2, b2x, zeros)


def _final_body(hp_ref, p_ref, out_ref):
    x = hp_ref[...]
    p = p_ref[...]
    agg = p[0] + p[1]
    iot = lax.broadcasted_iota(jnp.int32, x.shape, 1)
    sp = iot > 0

    def spat(v):
        return jnp.where(sp, v, 0.0)

    y = spat(x)
    a0 = agg[:, 0:1]
    mdv = jnp.sum(agg * agg, 1, keepdims=True) - 2.0 * a0 * a0
    normu = jnp.minimum(jnp.sqrt(jnp.clip(mdv, _EPS, None)), _MAX)
    th = jnp.maximum(normu, _MIN)
    rsp = (0.5 * (jnp.exp(th) + jnp.exp(-th))) * y \
        + (0.5 * (jnp.exp(th) - jnp.exp(-th))) * spat(agg) / th
    x0p = jnp.sqrt(jnp.clip(1.0 + jnp.sum(rsp * rsp, 1, keepdims=True), _EPS, None))
    yn = jnp.maximum(jnp.sqrt(jnp.sum(rsp * rsp, 1, keepdims=True)), _MIN)
    thh = jnp.maximum(x0p, 1.0 + _EPS)
    ach = jnp.log(thh + jnp.sqrt(jnp.clip(thh * thh - 1.0, 1e-15, None)))
    lt = ach * rsp / yn
    st = spat(lt / (1.0 + jnp.exp(-lt)))
    sn = jnp.maximum(jnp.sqrt(jnp.sum(st * st, 1, keepdims=True)), _MIN)
    xr = (0.5 * (jnp.exp(sn) - jnp.exp(-sn))) * st / sn
    ox0 = jnp.sqrt(jnp.clip(1.0 + jnp.sum(xr * xr, 1, keepdims=True), _EPS, None))
    out_ref[...] = jnp.where(iot == 0, ox0, xr)


def _stage3(hp, parts):
    grid = _N // _NB
    bs = pl.BlockSpec((_NB, _D), lambda i: (i, 0))
    return pl.pallas_call(
        _final_body,
        grid=(grid,),
        in_specs=[bs, pl.BlockSpec((2, _NB, _D), lambda i: (0, i, 0))],
        out_specs=bs,
        out_shape=jax.ShapeDtypeStruct((_N, _D), jnp.float32),
    )(hp, parts)


def kernel(h, edge_index, edge_attr, ln_g, ln_b, lin_W, lin_b,
           att_W1, att_b1, att_W2, att_b2):
    g128 = jnp.concatenate([jnp.ones((1,), jnp.float32), ln_g])[None, :]
    b128 = jnp.concatenate([jnp.zeros((1,), jnp.float32), ln_b])[None, :]
    linb = lin_b[None, :]
    w1r = att_W1[:, :_D]
    w1c = att_W1[:, _D:2 * _D]
    w1eT = att_W1[:, 2 * _D:].T
    b1 = att_b1[None, :]
    trow, tcol, hp = _stage1(h, g128, b128, lin_W, linb, w1r, w1c, b1)
    ee = _stage_ee(edge_attr.T, w1eT)
    row = edge_index[0].astype(jnp.int32)
    col = edge_index[1].astype(jnp.int32)
    w2 = att_W2.reshape(_D)
    b2x = jnp.ones((16,), jnp.float32) * att_b2
    zeros = jnp.zeros((_N, _D), jnp.float32)
    parts = _edge_sc(row, col, trow, tcol, ee, w2, b2x, zeros)
    return _stage3(hp, parts)


# speedup vs baseline: 1.4247x; 1.1175x over previous
import functools

import jax
import jax.numpy as jnp
from jax import lax
from jax.experimental import pallas as pl
from jax.experimental.pallas import tpu as pltpu
from jax.experimental.pallas import tpu_sc as plsc

_N = 10000
_E = 320000
_D = 128
_EPS = 1e-7
_MIN = 1e-15
_MAX = 1e6
_SQ50 = 7.0710678118654755

_NB = 1000
_EB = 16000
_CH = 32
_EPW = _E // 32


def _fsqrt(x):
    i = lax.bitcast_convert_type(x, jnp.int32)
    y = lax.bitcast_convert_type(jnp.int32(0x5F3759DF) - (i >> 1), jnp.float32)
    for _ in range(3):
        y = y * (1.5 - 0.5 * x * y * y)
    return x * y


def _flog(z):
    i = lax.bitcast_convert_type(z, jnp.int32)
    e = (i >> 23) - 127
    m = lax.bitcast_convert_type((i & 0x007FFFFF) | 0x3F800000, jnp.float32)
    big = m > 1.4142135
    m = jnp.where(big, 0.5 * m, m)
    ef = (e + big.astype(jnp.int32)).astype(jnp.float32)
    s = (m - 1.0) / (m + 1.0)
    s2 = s * s
    p = 2.0 * s * (1.0 + s2 * (1.0 / 3.0 + s2 * (0.2 + s2 * (1.0 / 7.0 + s2 * (1.0 / 9.0)))))
    return ef * 0.6931471805599453 + p


def _node_body(h_ref, g_ref, b_ref, w_ref, linb_ref, w1r_ref, w1c_ref, b1_ref,
               trow_ref, tcol_ref, hp_ref):
    X = h_ref[...]
    iot = lax.broadcasted_iota(jnp.int32, X.shape, 1)
    sp = iot > 0

    def spat(v):
        return jnp.where(sp, v, 0.0)

    def arcosh(x):
        return jnp.log(x + jnp.sqrt(jnp.clip(x * x - 1.0, 1e-15, None)))

    def logmap0(x):
        y = spat(x)
        yn = jnp.maximum(jnp.sqrt(jnp.sum(y * y, 1, keepdims=True)), _MIN)
        th = jnp.maximum(x[:, 0:1], 1.0 + _EPS)
        return arcosh(th) * y / yn

    def sinh(t):
        return 0.5 * (jnp.exp(t) - jnp.exp(-t))

    def cosh(t):
        return 0.5 * (jnp.exp(t) + jnp.exp(-t))

    def expmap0p(u):
        y = spat(u)
        yn = jnp.maximum(jnp.sqrt(jnp.sum(y * y, 1, keepdims=True)), _MIN)
        xr = sinh(yn) * y / yn
        x0 = jnp.sqrt(jnp.clip(1.0 + jnp.sum(xr * xr, 1, keepdims=True), _EPS, None))
        return jnp.where(iot == 0, x0, xr)

    def dotT(a, m_ref):
        return lax.dot_general(a, m_ref[...], (((1,), (1,)), ((), ())),
                               preferred_element_type=jnp.float32)

    ht = logmap0(X)
    mu = jnp.sum(ht, 1, keepdims=True) / (_D - 1)
    xc = ht - mu
    var = jnp.sum(spat(xc * xc), 1, keepdims=True) / (_D - 1)
    ln = xc / jnp.sqrt(var + 1e-5) * g_ref[...] + b_ref[...]
    h1 = expmap0p(spat(ln))
    u = logmap0(h1)
    h2 = expmap0p(dotT(u, w_ref))
    hb = expmap0p(spat(linb_ref[...]))
    ub = logmap0(hb)
    x0 = h2[:, 0:1]
    y = spat(h2)
    yn = jnp.maximum(jnp.sqrt(jnp.sum(y * y, 1, keepdims=True)), _MIN)
    yu = y / yn
    vv = jnp.where(iot == 0, -yn, (1.0 - x0) * yu)
    alpha = jnp.sum(yu * ub, 1, keepdims=True)
    w = ub - alpha * vv
    ux = jnp.sum(y * spat(w), 1, keepdims=True)
    w0 = ux / jnp.maximum(x0, _EPS)
    v = jnp.where(iot == 0, w0, w)
    mdv = jnp.sum(v * v, 1, keepdims=True) - 2.0 * w0 * w0
    normu = jnp.minimum(jnp.sqrt(jnp.clip(mdv, _EPS, None)), _MAX)
    th2 = jnp.maximum(normu, _MIN)
    rsp = cosh(th2) * y + sinh(th2) * spat(v) / th2
    x0p = jnp.sqrt(jnp.clip(1.0 + jnp.sum(rsp * rsp, 1, keepdims=True), _EPS, None))
    hp = jnp.where(iot == 0, x0p, rsp)
    xt = logmap0(hp)
    ar = dotT(xt, w1r_ref) + b1_ref[...]
    ac = dotT(xt, w1c_ref)
    trow_ref[...] = jnp.concatenate([hp, ar], axis=1)
    tcol_ref[...] = jnp.concatenate([hp, ac], axis=1)
    hp_ref[...] = hp


def _stage1(h, g128, b128, lin_W, linb, w1r, w1c, b1):
    grid = _N // _NB
    full = lambda shape: pl.BlockSpec(shape, lambda i: (0, 0))
    return pl.pallas_call(
        _node_body,
        grid=(grid,),
        in_specs=[
            pl.BlockSpec((_NB, _D), lambda i: (i, 0)),
            full((1, _D)), full((1, _D)), full((_D, _D)), full((1, _D)),
            full((_D, _D)), full((_D, _D)), full((1, _D)),
        ],
        out_specs=[
            pl.BlockSpec((_NB, 2 * _D), lambda i: (i, 0)),
            pl.BlockSpec((_NB, 2 * _D), lambda i: (i, 0)),
            pl.BlockSpec((_NB, _D), lambda i: (i, 0)),
        ],
        out_shape=[
            jax.ShapeDtypeStruct((_N, 2 * _D), jnp.float32),
            jax.ShapeDtypeStruct((_N, 2 * _D), jnp.float32),
            jax.ShapeDtypeStruct((_N, _D), jnp.float32),
        ],
    )(h, g128, b128, lin_W, linb, w1r, w1c, b1)


def _ee_body(eat_ref, w1e_ref, ee_ref):
    eat = eat_ref[...]
    w = w1e_ref[...]
    ee_ref[...] = lax.dot_general(eat, w, (((0,), (0,)), ((), ())),
                                  preferred_element_type=jnp.float32)


def _stage_ee(edge_attrT, w1eT):
    grid = _E // _EB
    return pl.pallas_call(
        _ee_body,
        grid=(grid,),
        in_specs=[
            pl.BlockSpec((4, _EB), lambda i: (0, i)),
            pl.BlockSpec((4, _D), lambda i: (0, 0)),
        ],
        out_specs=pl.BlockSpec((_EB, _D), lambda i: (i, 0)),
        out_shape=jax.ShapeDtypeStruct((_E, _D), jnp.float32),
    )(edge_attrT, w1eT)


def _edge_sc(idx, trow, tcol, ee, w2, b2x, zeros):
    mesh = plsc.VectorSubcoreMesh(core_axis_name="c", subcore_axis_name="s")
    rows_per_tile = (_N // 16) // 8 * 8
    tail0 = 16 * rows_per_tile
    tailn = _N - tail0

    @functools.partial(
        pl.kernel,
        out_type=jax.ShapeDtypeStruct((2, _N, _D), jnp.float32),
        mesh=mesh,
        scratch_types=[
            pltpu.VMEM((2, 2, _CH), jnp.int32),
            pltpu.VMEM((2, _CH, 2 * _D), jnp.float32),
            pltpu.VMEM((2, _CH, 2 * _D), jnp.float32),
            pltpu.VMEM((2, _CH, _D), jnp.float32),
            pltpu.VMEM((_CH, _D), jnp.float32),
            pltpu.VMEM((_D,), jnp.float32),
            pltpu.VMEM((16,), jnp.float32),
            pltpu.VMEM_SHARED((_N, _D), jnp.float32),
            pltpu.SemaphoreType.DMA,
            pltpu.SemaphoreType.DMA,
        ],
        compiler_params=pltpu.CompilerParams(use_tc_tiling_on_sc=False,
                                             needs_layout_passes=False),
    )
    def k(idx_hbm, trow_hbm, tcol_hbm, ee_hbm, w2_hbm, b2_hbm, z_hbm,
          out_hbm, idxv, rbuf, cbuf, eebuf, contrib, w2v, b2v, aggsh,
          sem0, sem1):
        c = lax.axis_index("c")
        s = lax.axis_index("s")
        widx = c * 16 + s
        r0 = s * rows_per_tile
        pltpu.sync_copy(z_hbm.at[pl.ds(r0, rows_per_tile)],
                        aggsh.at[pl.ds(r0, rows_per_tile)])

        if tailn:
            @pl.when(s == 0)
            def _():
                pltpu.sync_copy(z_hbm.at[pl.ds(tail0, tailn)],
                                aggsh.at[pl.ds(tail0, tailn)])

        pltpu.sync_copy(w2_hbm, w2v)
        pltpu.sync_copy(b2_hbm, b2v)
        plsc.subcore_barrier()
        iota = lax.iota(jnp.int32, 16)
        lane0 = (iota == 0).astype(jnp.float32)
        b2vec = b2v[...]
        w2reg = [w2v[pl.ds(16 * k, 16)] for k in range(8)]
        K8 = _D // 16

        def bc(sc):
            return jnp.full((16,), sc, jnp.float32)

        def rsum(v):
            return jnp.sum(v, axis=0)

        sems = (sem0, sem1)
        nfull = _EPW // _CH

        def base_of(ci):
            return widx * (nfull * _CH) + ci * _CH

        def issue(base, p):
            pltpu.sync_copy(idx_hbm.at[:, pl.ds(base, _CH)], idxv.at[p])
            pltpu.async_copy(trow_hbm.at[idxv.at[p, 0]], rbuf.at[p], sems[p])
            pltpu.async_copy(tcol_hbm.at[idxv.at[p, 1]], cbuf.at[p], sems[p])
            pltpu.async_copy(ee_hbm.at[pl.ds(base, _CH)], eebuf.at[p], sems[p])

        def drain(base, p):
            pltpu.make_async_copy(trow_hbm.at[idxv.at[p, 0]], rbuf.at[p], sems[p]).wait()
            pltpu.make_async_copy(tcol_hbm.at[idxv.at[p, 1]], cbuf.at[p], sems[p]).wait()
            pltpu.make_async_copy(ee_hbm.at[pl.ds(base, _CH)], eebuf.at[p], sems[p]).wait()

        def compute(p):
            @functools.partial(plsc.parallel_loop, 0, _CH, unroll=4)
            def _(e):
                xs = [rbuf[p, e, pl.ds(16 * k, 16)] for k in range(K8)]
                ys = [cbuf[p, e, pl.ds(16 * k, 16)] for k in range(K8)]
                dv = xs[0] * ys[0]
                for k in range(1, K8):
                    dv = dv + xs[k] * ys[k]
                s1 = bc(rsum(dv))
                x0 = bc(rsum(xs[0] * lane0))
                y0 = bc(rsum(ys[0] * lane0))
                pv = None
                for k in range(K8):
                    pre = rbuf[p, e, pl.ds(_D + 16 * k, 16)] \
                        + cbuf[p, e, pl.ds(_D + 16 * k, 16)] \
                        + eebuf[p, e, pl.ds(16 * k, 16)]
                    sa = pre / (1.0 + jnp.exp(-pre))
                    pv = sa * w2reg[k] if pv is None else pv + sa * w2reg[k]
                a2 = bc(rsum(pv))
                md = s1 - 2.0 * x0 * y0
                xy = jnp.minimum(md + 1.0, -_EPS) - 1.0
                th = jnp.maximum(-md, 1.0 + _EPS)
                dist = jnp.minimum(_flog(th + _fsqrt((th - 1.0) * (th + 1.0))), _SQ50)
                us = [ys[k] + xy * xs[k] for k in range(K8)]
                u0 = y0 + xy * x0
                qv = us[0] * us[0]
                tv = xs[0] * us[0]
                for k in range(1, K8):
                    qv = qv + us[k] * us[k]
                    tv = tv + xs[k] * us[k]
                s2 = bc(rsum(qv))
                s3 = bc(rsum(tv))
                normu = _fsqrt(jnp.maximum(s2 - 2.0 * u0 * u0, _EPS))
                att = 1.0 / (1.0 + jnp.exp(-(a2 + b2vec)))
                coef = att * dist / normu
                o0 = coef * (s3 - x0 * u0) / x0
                first = coef * us[0]
                contrib[e, pl.ds(0, 16)] = jnp.where(iota == 0, o0, first)
                for k in range(1, K8):
                    contrib[e, pl.ds(16 * k, 16)] = coef * us[k]

            pltpu.sync_copy(contrib, aggsh.at[idxv.at[p, 0]], add=True)

        main_span = 32 * nfull * _CH

        issue(base_of(0), 0)

        def outer(g, _):
            c0 = 2 * g
            issue(base_of(c0 + 1), 1)
            drain(base_of(c0), 0)
            compute(0)

            @pl.when(g < nfull // 2 - 1)
            def _():
                issue(base_of(c0 + 2), 0)

            drain(base_of(c0 + 1), 1)
            compute(1)
            return 0

        lax.fori_loop(0, nfull // 2, outer, 0)

        @pl.when(widx < (_E - main_span) // _CH)
        def _():
            base = main_span + widx * _CH
            issue(base, 0)
            drain(base, 0)
            compute(0)

        plsc.subcore_barrier()
        pltpu.sync_copy(aggsh.at[pl.ds(r0, rows_per_tile)],
                        out_hbm.at[c, pl.ds(r0, rows_per_tile)])

        if tailn:
            @pl.when(s == 0)
            def _():
                pltpu.sync_copy(aggsh.at[pl.ds(tail0, tailn)],
                                out_hbm.at[c, pl.ds(tail0, tailn)])

    return k(idx, trow, tcol, ee, w2, b2x, zeros)


def _final_body(hp_ref, p_ref, out_ref):
    x = hp_ref[...]
    p = p_ref[...]
    agg = p[0] + p[1]
    iot = lax.broadcasted_iota(jnp.int32, x.shape, 1)
    sp = iot > 0

    def spat(v):
        return jnp.where(sp, v, 0.0)

    y = spat(x)
    a0 = agg[:, 0:1]
    mdv = jnp.sum(agg * agg, 1, keepdims=True) - 2.0 * a0 * a0
    normu = jnp.minimum(jnp.sqrt(jnp.clip(mdv, _EPS, None)), _MAX)
    th = jnp.maximum(normu, _MIN)
    rsp = (0.5 * (jnp.exp(th) + jnp.exp(-th))) * y \
        + (0.5 * (jnp.exp(th) - jnp.exp(-th))) * spat(agg) / th
    x0p = jnp.sqrt(jnp.clip(1.0 + jnp.sum(rsp * rsp, 1, keepdims=True), _EPS, None))
    yn = jnp.maximum(jnp.sqrt(jnp.sum(rsp * rsp, 1, keepdims=True)), _MIN)
    thh = jnp.maximum(x0p, 1.0 + _EPS)
    ach = jnp.log(thh + jnp.sqrt(jnp.clip(thh * thh - 1.0, 1e-15, None)))
    lt = ach * rsp / yn
    st = spat(lt / (1.0 + jnp.exp(-lt)))
    sn = jnp.maximum(jnp.sqrt(jnp.sum(st * st, 1, keepdims=True)), _MIN)
    xr = (0.5 * (jnp.exp(sn) - jnp.exp(-sn))) * st / sn
    ox0 = jnp.sqrt(jnp.clip(1.0 + jnp.sum(xr * xr, 1, keepdims=True), _EPS, None))
    out_ref[...] = jnp.where(iot == 0, ox0, xr)


def _stage3(hp, parts):
    grid = _N // _NB
    bs = pl.BlockSpec((_NB, _D), lambda i: (i, 0))
    return pl.pallas_call(
        _final_body,
        grid=(grid,),
        in_specs=[bs, pl.BlockSpec((2, _NB, _D), lambda i: (0, i, 0))],
        out_specs=bs,
        out_shape=jax.ShapeDtypeStruct((_N, _D), jnp.float32),
    )(hp, parts)


def kernel(h, edge_index, edge_attr, ln_g, ln_b, lin_W, lin_b,
           att_W1, att_b1, att_W2, att_b2):
    g128 = jnp.concatenate([jnp.ones((1,), jnp.float32), ln_g])[None, :]
    b128 = jnp.concatenate([jnp.zeros((1,), jnp.float32), ln_b])[None, :]
    linb = lin_b[None, :]
    w1r = att_W1[:, :_D]
    w1c = att_W1[:, _D:2 * _D]
    w1eT = att_W1[:, 2 * _D:].T
    b1 = att_b1[None, :]
    trow, tcol, hp = _stage1(h, g128, b128, lin_W, linb, w1r, w1c, b1)
    ee = _stage_ee(edge_attr.T, w1eT)
    idx = edge_index.astype(jnp.int32)
    w2 = att_W2.reshape(_D)
    b2x = jnp.ones((16,), jnp.float32) * att_b2
    zeros = jnp.zeros((_N, _D), jnp.float32)
    parts = _edge_sc(idx, trow, tcol, ee, w2, b2x, zeros)
    return _stage3(hp, parts)
